# baseline (speedup 1.0000x reference)
import functools

import jax
import jax.numpy as jnp
from jax import lax
from jax.experimental import pallas as pl
from jax.experimental.pallas import tpu as pltpu
from jax.experimental.pallas import tpu_sc as plsc

N = 10000
D = 128
E = 320000

NC = 2
NS = 16
DH = D // NC
CH = 128
NCHUNK = 158
EPT = NCHUNK * CH
EPAD = EPT * NS
ROWS_PER_TILE = 640
ROWS = ROWS_PER_TILE * NS
DEGW = 16

_mesh = plsc.VectorSubcoreMesh(core_axis_name="c", subcore_axis_name="s")


@functools.partial(
    pl.kernel,
    mesh=_mesh,
    out_type=[
        jax.ShapeDtypeStruct((NC, ROWS, DH), jnp.bfloat16),
        jax.ShapeDtypeStruct((NC, ROWS, DEGW), jnp.float32),
    ],
    scratch_types=[
        pltpu.VMEM((NCHUNK, CH), jnp.int32),
        pltpu.VMEM((NCHUNK, CH), jnp.int32),
        pltpu.VMEM((CH, DH), jnp.bfloat16),
        pltpu.VMEM((CH, DH), jnp.bfloat16),
        pltpu.VMEM((CH, DEGW), jnp.float32),
        pltpu.VMEM((CH, DEGW), jnp.float32),
        pltpu.VMEM_SHARED((ROWS, DH), jnp.bfloat16),
        pltpu.VMEM_SHARED((ROWS, DEGW), jnp.float32),
        pltpu.SemaphoreType.DMA,
        pltpu.SemaphoreType.DMA,
    ],
    compiler_params=pltpu.CompilerParams(use_tc_tiling_on_sc=False),
)
def _sc_aggregate(x_hbm, src_hbm, dst_hbm, zbf_hbm, z16_hbm, ones_hbm,
                  acc_out, deg_out,
                  srcv, dstv, bufa, bufb, onesv, z16v, acc_sh, deg_sh,
                  sema, semb):
    c = lax.axis_index("c")
    s = lax.axis_index("s")
    rbase = s * ROWS_PER_TILE

    pltpu.sync_copy(zbf_hbm, bufa)
    pltpu.sync_copy(z16_hbm, z16v)
    pltpu.sync_copy(ones_hbm, onesv)
    pltpu.sync_copy(src_hbm.at[c, s], srcv)
    pltpu.sync_copy(dst_hbm.at[s], dstv)

    for r in range(ROWS_PER_TILE // CH):
        pltpu.sync_copy(bufa, acc_sh.at[pl.ds(rbase + r * CH, CH)])
        pltpu.sync_copy(z16v, deg_sh.at[pl.ds(rbase + r * CH, CH)])
    plsc.subcore_barrier()

    pltpu.async_copy(x_hbm.at[srcv.at[0]], bufa, sema)
    pltpu.async_copy(x_hbm.at[srcv.at[1]], bufb, semb)

    def body(g, car):
        pltpu.make_async_copy(x_hbm.at[srcv.at[g]], bufa, sema).wait()
        pltpu.sync_copy(bufa, acc_sh.at[dstv.at[g]], add=True)
        pltpu.sync_copy(onesv, deg_sh.at[dstv.at[g]], add=True)

        @pl.when(g + 2 < NCHUNK)
        def _start_a():
            pltpu.async_copy(x_hbm.at[srcv.at[g + 2]], bufa, sema)

        pltpu.make_async_copy(x_hbm.at[srcv.at[g + 1]], bufb, semb).wait()
        pltpu.sync_copy(bufb, acc_sh.at[dstv.at[g + 1]], add=True)
        pltpu.sync_copy(onesv, deg_sh.at[dstv.at[g + 1]], add=True)

        @pl.when(g + 3 < NCHUNK)
        def _start_b():
            pltpu.async_copy(x_hbm.at[srcv.at[g + 3]], bufb, semb)

        return car

    lax.fori_loop(0, NCHUNK // 2, lambda i, car: body(i * 2, car), 0,
                  unroll=False)

    plsc.subcore_barrier()
    pltpu.sync_copy(acc_sh.at[pl.ds(rbase, ROWS_PER_TILE)],
                    acc_out.at[c, pl.ds(rbase, ROWS_PER_TILE)])
    pltpu.sync_copy(deg_sh.at[pl.ds(rbase, ROWS_PER_TILE)],
                    deg_out.at[c, pl.ds(rbase, ROWS_PER_TILE)])


def _tc_body(x_ref, a0_ref, a1_ref, d0_ref, d1_ref,
             whp_ref, bhp_ref, wlp_ref, blp_ref, wid_ref, bid_ref,
             wh_ref, bh_ref, wl_ref, bl_ref, wi_ref, bi_ref,
             out_ref):
    x = x_ref[...]
    deg = d0_ref[:, 0:1]
    acc = jnp.concatenate([a0_ref[...], a1_ref[...]],
                          axis=1).astype(jnp.float32)
    agg = acc / jnp.maximum(deg, 1.0)
    h_hp = jnp.maximum(
        jnp.dot(x - agg, whp_ref[...], preferred_element_type=jnp.float32)
        + bhp_ref[...], 0.0)
    h_lp = jnp.maximum(
        jnp.dot(agg, wlp_ref[...], preferred_element_type=jnp.float32)
        + blp_ref[...], 0.0)
    h_id = jnp.maximum(
        jnp.dot(x, wid_ref[...], preferred_element_type=jnp.float32)
        + bid_ref[...], 0.0)
    a_h = jax.nn.sigmoid(
        jnp.sum(h_hp * wh_ref[...], axis=1, keepdims=True) + bh_ref[...])
    a_l = jax.nn.sigmoid(
        jnp.sum(h_lp * wl_ref[...], axis=1, keepdims=True) + bl_ref[...])
    a_i = jax.nn.sigmoid(
        jnp.sum(h_id * wi_ref[...], axis=1, keepdims=True) + bi_ref[...])
    out_ref[...] = a_h * h_hp + a_l * h_lp + a_i * h_id


def kernel(x, edge_index, W_hp, b_hp, W_lp, b_lp, W_id, b_id,
           wh, bh, wl, bl, wi, bi):
    src = edge_index[0]
    dst = edge_index[1]
    pad = EPAD - E
    src_p = jnp.concatenate([src, jnp.zeros((pad,), jnp.int32)])
    src_p = jnp.stack([src_p, src_p + N]).reshape(NC, NS, NCHUNK, CH)
    dst_p = jnp.concatenate(
        [dst, jnp.full((pad,), N, jnp.int32)]).reshape(NS, NCHUNK, CH)
    x_halves = jnp.concatenate(
        [x[:, :DH], x[:, DH:]], axis=0).astype(jnp.bfloat16)
    zbf = jnp.zeros((CH, DH), jnp.bfloat16)
    z16 = jnp.zeros((CH, DEGW), jnp.float32)
    ones16 = jnp.ones((CH, DEGW), jnp.float32)

    acc, deg = _sc_aggregate(x_halves, src_p, dst_p, zbf, z16, ones16)

    rb = 1000
    grid = (N // rb,)
    row_spec = pl.BlockSpec((rb, D), lambda i: (i, 0))
    half_spec = pl.BlockSpec((rb, DH), lambda i: (i, 0))
    deg_spec = pl.BlockSpec((rb, DEGW), lambda i: (i, 0))
    full = lambda shape: pl.BlockSpec(shape, lambda i: (0,) * len(shape))
    out = pl.pallas_call(
        _tc_body,
        grid=grid,
        in_specs=[
            row_spec, half_spec, half_spec, deg_spec, deg_spec,
            full((D, D)), full((1, D)),
            full((D, D)), full((1, D)),
            full((D, D)), full((1, D)),
            full((1, D)), full((1, 1)),
            full((1, D)), full((1, 1)),
            full((1, D)), full((1, 1)),
        ],
        out_specs=row_spec,
        out_shape=jax.ShapeDtypeStruct((N, D), jnp.float32),
    )(x, acc[0], acc[1], deg[0], deg[1],
      W_hp, b_hp.reshape(1, D), W_lp, b_lp.reshape(1, D),
      W_id, b_id.reshape(1, D),
      wh.reshape(1, D), bh.reshape(1, 1),
      wl.reshape(1, D), bl.reshape(1, 1),
      wi.reshape(1, D), bi.reshape(1, 1))
    return out


# speedup vs baseline: 1.1435x; 1.1435x over previous
import functools

import jax
import jax.numpy as jnp
from jax import lax
from jax.experimental import pallas as pl
from jax.experimental.pallas import tpu as pltpu
from jax.experimental.pallas import tpu_sc as plsc

N = 10000
D = 128
E = 320000

NC = 2
NS = 16
DH = D // NC
CH = 128
NCHUNK = 158
EPT = NCHUNK * CH
EPAD = EPT * NS
ROWS_PER_TILE = 640
ROWS = ROWS_PER_TILE * NS
DEGW = 16

_mesh = plsc.VectorSubcoreMesh(core_axis_name="c", subcore_axis_name="s")


@functools.partial(
    pl.kernel,
    mesh=_mesh,
    out_type=[
        jax.ShapeDtypeStruct((NC, ROWS, DH), jnp.bfloat16),
        jax.ShapeDtypeStruct((NC, ROWS, DEGW), jnp.float32),
    ],
    scratch_types=[
        pltpu.VMEM((NCHUNK, CH), jnp.int32),
        pltpu.VMEM((NCHUNK, CH), jnp.int32),
        pltpu.VMEM((CH, DH), jnp.bfloat16),
        pltpu.VMEM((CH, DH), jnp.bfloat16),
        pltpu.VMEM((CH, DEGW), jnp.float32),
        pltpu.VMEM((CH, DEGW), jnp.float32),
        pltpu.VMEM_SHARED((ROWS, DH), jnp.bfloat16),
        pltpu.VMEM_SHARED((ROWS, DEGW), jnp.float32),
        pltpu.SemaphoreType.DMA,
        pltpu.SemaphoreType.DMA,
    ],
    compiler_params=pltpu.CompilerParams(use_tc_tiling_on_sc=False),
)
def _sc_aggregate(x_hbm, src_hbm, dst_hbm, zbf_hbm, z16_hbm, ones_hbm,
                  acc_out, deg_out,
                  srcv, dstv, bufa, bufb, onesv, z16v, acc_sh, deg_sh,
                  sema, semb):
    c = lax.axis_index("c")
    s = lax.axis_index("s")
    rbase = s * ROWS_PER_TILE

    pltpu.sync_copy(zbf_hbm, bufa)
    pltpu.sync_copy(z16_hbm, z16v)
    pltpu.sync_copy(ones_hbm, onesv)
    pltpu.sync_copy(src_hbm.at[c, s], srcv)
    pltpu.sync_copy(dst_hbm.at[s], dstv)

    for r in range(ROWS_PER_TILE // CH):
        pltpu.sync_copy(bufa, acc_sh.at[pl.ds(rbase + r * CH, CH)])
        pltpu.sync_copy(z16v, deg_sh.at[pl.ds(rbase + r * CH, CH)])
    plsc.subcore_barrier()

    pltpu.async_copy(x_hbm.at[srcv.at[0]], bufa, sema)
    pltpu.async_copy(x_hbm.at[srcv.at[1]], bufb, semb)

    def body(g, car):
        pltpu.make_async_copy(x_hbm.at[srcv.at[g]], bufa, sema).wait()
        pltpu.sync_copy(bufa, acc_sh.at[dstv.at[g]], add=True)
        pltpu.sync_copy(onesv, deg_sh.at[dstv.at[g]], add=True)

        @pl.when(g + 2 < NCHUNK)
        def _start_a():
            pltpu.async_copy(x_hbm.at[srcv.at[g + 2]], bufa, sema)

        pltpu.make_async_copy(x_hbm.at[srcv.at[g + 1]], bufb, semb).wait()
        pltpu.sync_copy(bufb, acc_sh.at[dstv.at[g + 1]], add=True)
        pltpu.sync_copy(onesv, deg_sh.at[dstv.at[g + 1]], add=True)

        @pl.when(g + 3 < NCHUNK)
        def _start_b():
            pltpu.async_copy(x_hbm.at[srcv.at[g + 3]], bufb, semb)

        return car

    lax.fori_loop(0, NCHUNK // 2, lambda i, car: body(i * 2, car), 0,
                  unroll=False)

    plsc.subcore_barrier()
    pltpu.sync_copy(acc_sh.at[pl.ds(rbase, ROWS_PER_TILE)],
                    acc_out.at[c, pl.ds(rbase, ROWS_PER_TILE)])
    pltpu.sync_copy(deg_sh.at[pl.ds(rbase, ROWS_PER_TILE)],
                    deg_out.at[c, pl.ds(rbase, ROWS_PER_TILE)])


def _prep_x_body(x_ref, o_ref):
    xb = x_ref[...]
    o_ref[0] = xb[:, :DH].astype(jnp.bfloat16)
    o_ref[1] = xb[:, DH:].astype(jnp.bfloat16)


def _tc_body(x_ref, a0_ref, a1_ref, d0_ref, d1_ref,
             whp_ref, bhp_ref, wlp_ref, blp_ref, wid_ref, bid_ref,
             wh_ref, bh_ref, wl_ref, bl_ref, wi_ref, bi_ref,
             out_ref):
    x = x_ref[...]
    deg = d0_ref[:, 0:1]
    acc = jnp.concatenate([a0_ref[...], a1_ref[...]],
                          axis=1).astype(jnp.float32)
    agg = acc / jnp.maximum(deg, 1.0)
    h_hp = jnp.maximum(
        jnp.dot(x - agg, whp_ref[...], preferred_element_type=jnp.float32)
        + bhp_ref[...], 0.0)
    h_lp = jnp.maximum(
        jnp.dot(agg, wlp_ref[...], preferred_element_type=jnp.float32)
        + blp_ref[...], 0.0)
    h_id = jnp.maximum(
        jnp.dot(x, wid_ref[...], preferred_element_type=jnp.float32)
        + bid_ref[...], 0.0)
    a_h = jax.nn.sigmoid(
        jnp.sum(h_hp * wh_ref[...], axis=1, keepdims=True) + bh_ref[...])
    a_l = jax.nn.sigmoid(
        jnp.sum(h_lp * wl_ref[...], axis=1, keepdims=True) + bl_ref[...])
    a_i = jax.nn.sigmoid(
        jnp.sum(h_id * wi_ref[...], axis=1, keepdims=True) + bi_ref[...])
    out_ref[...] = a_h * h_hp + a_l * h_lp + a_i * h_id


def kernel(x, edge_index, W_hp, b_hp, W_lp, b_lp, W_id, b_id,
           wh, bh, wl, bl, wi, bi):
    src = edge_index[0]
    dst = edge_index[1]
    pad = EPAD - E
    src_p = jnp.concatenate([src, jnp.zeros((pad,), jnp.int32)])
    src_p = jnp.stack([src_p, src_p + N]).reshape(NC, NS, NCHUNK, CH)
    dst_p = jnp.concatenate(
        [dst, jnp.full((pad,), N, jnp.int32)]).reshape(NS, NCHUNK, CH)
    pb = 1000
    x_halves = pl.pallas_call(
        _prep_x_body,
        grid=(N // pb,),
        in_specs=[pl.BlockSpec((pb, D), lambda i: (i, 0))],
        out_specs=pl.BlockSpec((NC, pb, DH), lambda i: (0, i, 0)),
        out_shape=jax.ShapeDtypeStruct((NC, N, DH), jnp.bfloat16),
    )(x).reshape(NC * N, DH)
    zbf = jnp.zeros((CH, DH), jnp.bfloat16)
    z16 = jnp.zeros((CH, DEGW), jnp.float32)
    ones16 = jnp.ones((CH, DEGW), jnp.float32)

    acc, deg = _sc_aggregate(x_halves, src_p, dst_p, zbf, z16, ones16)

    rb = 1000
    grid = (N // rb,)
    row_spec = pl.BlockSpec((rb, D), lambda i: (i, 0))
    half_spec = pl.BlockSpec((rb, DH), lambda i: (i, 0))
    deg_spec = pl.BlockSpec((rb, DEGW), lambda i: (i, 0))
    full = lambda shape: pl.BlockSpec(shape, lambda i: (0,) * len(shape))
    out = pl.pallas_call(
        _tc_body,
        grid=grid,
        in_specs=[
            row_spec, half_spec, half_spec, deg_spec, deg_spec,
            full((D, D)), full((1, D)),
            full((D, D)), full((1, D)),
            full((D, D)), full((1, D)),
            full((1, D)), full((1, 1)),
            full((1, D)), full((1, 1)),
            full((1, D)), full((1, 1)),
        ],
        out_specs=row_spec,
        out_shape=jax.ShapeDtypeStruct((N, D), jnp.float32),
    )(x, acc[0], acc[1], deg[0], deg[1],
      W_hp, b_hp.reshape(1, D), W_lp, b_lp.reshape(1, D),
      W_id, b_id.reshape(1, D),
      wh.reshape(1, D), bh.reshape(1, 1),
      wl.reshape(1, D), bl.reshape(1, 1),
      wi.reshape(1, D), bi.reshape(1, 1))
    return out


# speedup vs baseline: 1.1964x; 1.0463x over previous
import functools

import jax
import jax.numpy as jnp
from jax import lax
from jax.experimental import pallas as pl
from jax.experimental.pallas import tpu as pltpu
from jax.experimental.pallas import tpu_sc as plsc

N = 10000
D = 128
E = 320000

NC = 2
NS = 16
DH = D // NC
CH = 128
NCHUNK = 158
EPT = NCHUNK * CH
EPAD = EPT * NS
ROWS_PER_TILE = 640
ROWS = ROWS_PER_TILE * NS
DEGW = 16

_mesh = plsc.VectorSubcoreMesh(core_axis_name="c", subcore_axis_name="s")


@functools.partial(
    pl.kernel,
    mesh=_mesh,
    out_type=[
        jax.ShapeDtypeStruct((NC, ROWS, DH), jnp.bfloat16),
        jax.ShapeDtypeStruct((NC, ROWS, DEGW), jnp.float32),
    ],
    scratch_types=[
        pltpu.VMEM((NCHUNK, CH), jnp.int32),
        pltpu.VMEM((NCHUNK, CH), jnp.int32),
        pltpu.VMEM((CH, DH), jnp.bfloat16),
        pltpu.VMEM((CH, DH), jnp.bfloat16),
        pltpu.VMEM((CH, DEGW), jnp.float32),
        pltpu.VMEM((CH, DEGW), jnp.float32),
        pltpu.VMEM_SHARED((ROWS, DH), jnp.bfloat16),
        pltpu.VMEM_SHARED((ROWS, DEGW), jnp.float32),
        pltpu.SemaphoreType.DMA,
        pltpu.SemaphoreType.DMA,
    ],
    compiler_params=pltpu.CompilerParams(use_tc_tiling_on_sc=False),
)
def _sc_aggregate(x_hbm, src_hbm, dst_hbm, zbf_hbm, z16_hbm, ones_hbm,
                  acc_out, deg_out,
                  srcv, dstv, bufa, bufb, onesv, z16v, acc_sh, deg_sh,
                  sema, semb):
    c = lax.axis_index("c")
    s = lax.axis_index("s")
    rbase = s * ROWS_PER_TILE

    pltpu.sync_copy(zbf_hbm, bufa)
    pltpu.sync_copy(z16_hbm, z16v)
    pltpu.sync_copy(ones_hbm, onesv)
    pltpu.sync_copy(src_hbm.at[s], srcv)
    pltpu.sync_copy(dst_hbm.at[s], dstv)
    xview = x_hbm.at[pl.ds(c * N, N)]

    for r in range(ROWS_PER_TILE // CH):
        pltpu.sync_copy(bufa, acc_sh.at[pl.ds(rbase + r * CH, CH)])
        pltpu.sync_copy(z16v, deg_sh.at[pl.ds(rbase + r * CH, CH)])
    plsc.subcore_barrier()

    pltpu.async_copy(xview.at[srcv.at[0]], bufa, sema)
    pltpu.async_copy(xview.at[srcv.at[1]], bufb, semb)

    def body(g, car):
        pltpu.make_async_copy(xview.at[srcv.at[g]], bufa, sema).wait()
        pltpu.sync_copy(bufa, acc_sh.at[dstv.at[g]], add=True)
        pltpu.sync_copy(onesv, deg_sh.at[dstv.at[g]], add=True)

        @pl.when(g + 2 < NCHUNK)
        def _start_a():
            pltpu.async_copy(xview.at[srcv.at[g + 2]], bufa, sema)

        pltpu.make_async_copy(xview.at[srcv.at[g + 1]], bufb, semb).wait()
        pltpu.sync_copy(bufb, acc_sh.at[dstv.at[g + 1]], add=True)
        pltpu.sync_copy(onesv, deg_sh.at[dstv.at[g + 1]], add=True)

        @pl.when(g + 3 < NCHUNK)
        def _start_b():
            pltpu.async_copy(xview.at[srcv.at[g + 3]], bufb, semb)

        return car

    lax.fori_loop(0, NCHUNK // 2, lambda i, car: body(i * 2, car), 0,
                  unroll=False)

    plsc.subcore_barrier()
    pltpu.sync_copy(acc_sh.at[pl.ds(rbase, ROWS_PER_TILE)],
                    acc_out.at[c, pl.ds(rbase, ROWS_PER_TILE)])
    pltpu.sync_copy(deg_sh.at[pl.ds(rbase, ROWS_PER_TILE)],
                    deg_out.at[c, pl.ds(rbase, ROWS_PER_TILE)])


def _prep_e_body(e0_ref, e1_ref, s_ref, d_ref):
    nrow = E // CH
    npad = NS * NCHUNK - nrow
    s_ref[...] = jnp.concatenate(
        [e0_ref[0], jnp.zeros((npad, CH), jnp.int32)], axis=0)
    d_ref[...] = jnp.concatenate(
        [e1_ref[0], jnp.full((npad, CH), N, jnp.int32)], axis=0)


def _prep_x_body(x_ref, o_ref):
    xb = x_ref[...]
    o_ref[0] = xb[:, :DH].astype(jnp.bfloat16)
    o_ref[1] = xb[:, DH:].astype(jnp.bfloat16)


def _tc_body(x_ref, a0_ref, a1_ref, d0_ref, d1_ref,
             whp_ref, bhp_ref, wlp_ref, blp_ref, wid_ref, bid_ref,
             wh_ref, bh_ref, wl_ref, bl_ref, wi_ref, bi_ref,
             out_ref):
    x = x_ref[...]
    deg = d0_ref[:, 0:1]
    acc = jnp.concatenate([a0_ref[...], a1_ref[...]],
                          axis=1).astype(jnp.float32)
    agg = acc / jnp.maximum(deg, 1.0)
    h_hp = jnp.maximum(
        jnp.dot(x - agg, whp_ref[...], preferred_element_type=jnp.float32)
        + bhp_ref[...], 0.0)
    h_lp = jnp.maximum(
        jnp.dot(agg, wlp_ref[...], preferred_element_type=jnp.float32)
        + blp_ref[...], 0.0)
    h_id = jnp.maximum(
        jnp.dot(x, wid_ref[...], preferred_element_type=jnp.float32)
        + bid_ref[...], 0.0)
    a_h = jax.nn.sigmoid(
        jnp.sum(h_hp * wh_ref[...], axis=1, keepdims=True) + bh_ref[...])
    a_l = jax.nn.sigmoid(
        jnp.sum(h_lp * wl_ref[...], axis=1, keepdims=True) + bl_ref[...])
    a_i = jax.nn.sigmoid(
        jnp.sum(h_id * wi_ref[...], axis=1, keepdims=True) + bi_ref[...])
    out_ref[...] = a_h * h_hp + a_l * h_lp + a_i * h_id


def kernel(x, edge_index, W_hp, b_hp, W_lp, b_lp, W_id, b_id,
           wh, bh, wl, bl, wi, bi):
    nrow = E // CH
    erows = NS * NCHUNK
    e_r = edge_index.reshape(2, nrow, CH)
    src_p, dst_p = pl.pallas_call(
        _prep_e_body,
        grid=(1,),
        in_specs=[pl.BlockSpec((1, nrow, CH), lambda i: (0, 0, 0)),
                  pl.BlockSpec((1, nrow, CH), lambda i: (1, 0, 0))],
        out_specs=[pl.BlockSpec((erows, CH), lambda i: (0, 0))] * 2,
        out_shape=[jax.ShapeDtypeStruct((erows, CH), jnp.int32)] * 2,
    )(e_r, e_r)
    src_p = src_p.reshape(NS, NCHUNK, CH)
    dst_p = dst_p.reshape(NS, NCHUNK, CH)
    pb = 1000
    x_halves = pl.pallas_call(
        _prep_x_body,
        grid=(N // pb,),
        in_specs=[pl.BlockSpec((pb, D), lambda i: (i, 0))],
        out_specs=pl.BlockSpec((NC, pb, DH), lambda i: (0, i, 0)),
        out_shape=jax.ShapeDtypeStruct((NC, N, DH), jnp.bfloat16),
    )(x).reshape(NC * N, DH)
    zbf = jnp.zeros((CH, DH), jnp.bfloat16)
    z16 = jnp.zeros((CH, DEGW), jnp.float32)
    ones16 = jnp.ones((CH, DEGW), jnp.float32)

    acc, deg = _sc_aggregate(x_halves, src_p, dst_p, zbf, z16, ones16)

    rb = 1000
    grid = (N // rb,)
    row_spec = pl.BlockSpec((rb, D), lambda i: (i, 0))
    half_spec = pl.BlockSpec((rb, DH), lambda i: (i, 0))
    deg_spec = pl.BlockSpec((rb, DEGW), lambda i: (i, 0))
    full = lambda shape: pl.BlockSpec(shape, lambda i: (0,) * len(shape))
    out = pl.pallas_call(
        _tc_body,
        grid=grid,
        in_specs=[
            row_spec, half_spec, half_spec, deg_spec, deg_spec,
            full((D, D)), full((1, D)),
            full((D, D)), full((1, D)),
            full((D, D)), full((1, D)),
            full((1, D)), full((1, 1)),
            full((1, D)), full((1, 1)),
            full((1, D)), full((1, 1)),
        ],
        out_specs=row_spec,
        out_shape=jax.ShapeDtypeStruct((N, D), jnp.float32),
    )(x, acc[0], acc[1], deg[0], deg[1],
      W_hp, b_hp.reshape(1, D), W_lp, b_lp.reshape(1, D),
      W_id, b_id.reshape(1, D),
      wh.reshape(1, D), bh.reshape(1, 1),
      wl.reshape(1, D), bl.reshape(1, 1),
      wi.reshape(1, D), bi.reshape(1, 1))
    return out


# speedup vs baseline: 1.2419x; 1.0381x over previous
import functools

import jax
import jax.numpy as jnp
from jax import lax
from jax.experimental import pallas as pl
from jax.experimental.pallas import tpu as pltpu
from jax.experimental.pallas import tpu_sc as plsc

N = 10000
D = 128
E = 320000

NC = 2
NS = 16
DH = D // NC
CH = 128
NCHUNK = 158
EPT = NCHUNK * CH
EPAD = EPT * NS
ROWS_PER_TILE = 640
ROWS = ROWS_PER_TILE * NS
DEGW = 16

_mesh = plsc.VectorSubcoreMesh(core_axis_name="c", subcore_axis_name="s")


@functools.partial(
    pl.kernel,
    mesh=_mesh,
    out_type=[
        jax.ShapeDtypeStruct((NC, ROWS, DH), jnp.bfloat16),
        jax.ShapeDtypeStruct((ROWS, DEGW), jnp.float32),
    ],
    scratch_types=[
        pltpu.VMEM((NCHUNK, CH), jnp.int32),
        pltpu.VMEM((NCHUNK, CH), jnp.int32),
        pltpu.VMEM((CH, DH), jnp.bfloat16),
        pltpu.VMEM((CH, DH), jnp.bfloat16),
        pltpu.VMEM((CH, DEGW), jnp.float32),
        pltpu.VMEM((CH, DEGW), jnp.float32),
        pltpu.VMEM_SHARED((ROWS, DH), jnp.bfloat16),
        pltpu.VMEM_SHARED((ROWS, DEGW), jnp.float32),
        pltpu.SemaphoreType.DMA,
        pltpu.SemaphoreType.DMA,
    ],
    compiler_params=pltpu.CompilerParams(use_tc_tiling_on_sc=False),
)
def _sc_aggregate(x_hbm, edge_hbm, zbf_hbm, z16_hbm, ones_hbm,
                  acc_out, deg_out,
                  srcv, dstv, bufa, bufb, onesv, z16v, acc_sh, deg_sh,
                  sema, semb):
    c = lax.axis_index("c")
    s = lax.axis_index("s")
    rbase = s * ROWS_PER_TILE

    pltpu.sync_copy(zbf_hbm, bufa)
    pltpu.sync_copy(z16_hbm, z16v)
    pltpu.sync_copy(ones_hbm, onesv)
    pltpu.sync_copy(edge_hbm.at[0, s], srcv)
    pltpu.sync_copy(edge_hbm.at[1, s], dstv)
    xview = x_hbm.at[c]

    for r in range(ROWS_PER_TILE // CH):
        pltpu.sync_copy(bufa, acc_sh.at[pl.ds(rbase + r * CH, CH)])
        pltpu.sync_copy(z16v, deg_sh.at[pl.ds(rbase + r * CH, CH)])
    plsc.subcore_barrier()

    pltpu.async_copy(xview.at[srcv.at[0]], bufa, sema)
    pltpu.async_copy(xview.at[srcv.at[1]], bufb, semb)

    def body(g, car):
        pltpu.make_async_copy(xview.at[srcv.at[g]], bufa, sema).wait()
        pltpu.sync_copy(bufa, acc_sh.at[dstv.at[g]], add=True)
        pltpu.sync_copy(onesv, deg_sh.at[dstv.at[g]], add=True)

        @pl.when(g + 2 < NCHUNK)
        def _start_a():
            pltpu.async_copy(xview.at[srcv.at[g + 2]], bufa, sema)

        pltpu.make_async_copy(xview.at[srcv.at[g + 1]], bufb, semb).wait()
        pltpu.sync_copy(bufb, acc_sh.at[dstv.at[g + 1]], add=True)
        pltpu.sync_copy(onesv, deg_sh.at[dstv.at[g + 1]], add=True)

        @pl.when(g + 3 < NCHUNK)
        def _start_b():
            pltpu.async_copy(xview.at[srcv.at[g + 3]], bufb, semb)

        return car

    lax.fori_loop(0, NCHUNK // 2, lambda i, car: body(i * 2, car), 0,
                  unroll=False)

    plsc.subcore_barrier()
    pltpu.sync_copy(acc_sh.at[pl.ds(rbase, ROWS_PER_TILE)],
                    acc_out.at[c, pl.ds(rbase, ROWS_PER_TILE)])
    @pl.when(c == 0)
    def _flush_deg():
        pltpu.sync_copy(deg_sh.at[pl.ds(rbase, ROWS_PER_TILE)],
                        deg_out.at[pl.ds(rbase, ROWS_PER_TILE)])


def _prep_x_body(x_ref, o_ref):
    xb = x_ref[...]
    o_ref[0] = xb[:, :DH].astype(jnp.bfloat16)
    o_ref[1] = xb[:, DH:].astype(jnp.bfloat16)


def _tc_body(x_ref, a0_ref, a1_ref, d0_ref,
             whp_ref, bhp_ref, wlp_ref, blp_ref, wid_ref, bid_ref,
             wh_ref, bh_ref, wl_ref, bl_ref, wi_ref, bi_ref,
             out_ref):
    x = x_ref[...]
    deg = d0_ref[:, 0:1]
    acc = jnp.concatenate([a0_ref[...], a1_ref[...]],
                          axis=1).astype(jnp.float32)
    agg = acc / jnp.maximum(deg, 1.0)
    bf = jnp.bfloat16
    h_hp = jnp.maximum(
        jnp.dot((x - agg).astype(bf), whp_ref[...].astype(bf),
                preferred_element_type=jnp.float32) + bhp_ref[...], 0.0)
    h_lp = jnp.maximum(
        jnp.dot(agg.astype(bf), wlp_ref[...].astype(bf),
                preferred_element_type=jnp.float32) + blp_ref[...], 0.0)
    h_id = jnp.maximum(
        jnp.dot(x.astype(bf), wid_ref[...].astype(bf),
                preferred_element_type=jnp.float32) + bid_ref[...], 0.0)
    a_h = jax.nn.sigmoid(
        jnp.sum(h_hp * wh_ref[...], axis=1, keepdims=True) + bh_ref[...])
    a_l = jax.nn.sigmoid(
        jnp.sum(h_lp * wl_ref[...], axis=1, keepdims=True) + bl_ref[...])
    a_i = jax.nn.sigmoid(
        jnp.sum(h_id * wi_ref[...], axis=1, keepdims=True) + bi_ref[...])
    out_ref[...] = a_h * h_hp + a_l * h_lp + a_i * h_id


def kernel(x, edge_index, W_hp, b_hp, W_lp, b_lp, W_id, b_id,
           wh, bh, wl, bl, wi, bi):
    pad = EPAD - E
    pad_block = jnp.concatenate(
        [jnp.zeros((1, pad), jnp.int32), jnp.full((1, pad), N, jnp.int32)])
    edge_p = jnp.concatenate([edge_index, pad_block], axis=1)
    edge_p = edge_p.reshape(2, NS, NCHUNK, CH)
    pb = 1000
    x_halves = pl.pallas_call(
        _prep_x_body,
        grid=(N // pb,),
        in_specs=[pl.BlockSpec((pb, D), lambda i: (i, 0))],
        out_specs=pl.BlockSpec((NC, pb, DH), lambda i: (0, i, 0)),
        out_shape=jax.ShapeDtypeStruct((NC, N, DH), jnp.bfloat16),
    )(x)
    zbf = jnp.zeros((CH, DH), jnp.bfloat16)
    z16 = jnp.zeros((CH, DEGW), jnp.float32)
    ones16 = jnp.ones((CH, DEGW), jnp.float32)

    acc, deg = _sc_aggregate(x_halves, edge_p, zbf, z16, ones16)

    rb = 1000
    grid = (N // rb,)
    row_spec = pl.BlockSpec((rb, D), lambda i: (i, 0))
    half_spec = pl.BlockSpec((rb, DH), lambda i: (i, 0))
    deg_spec = pl.BlockSpec((rb, DEGW), lambda i: (i, 0))
    full = lambda shape: pl.BlockSpec(shape, lambda i: (0,) * len(shape))
    out = pl.pallas_call(
        _tc_body,
        grid=grid,
        in_specs=[
            row_spec, half_spec, half_spec, deg_spec,
            full((D, D)), full((1, D)),
            full((D, D)), full((1, D)),
            full((D, D)), full((1, D)),
            full((1, D)), full((1, 1)),
            full((1, D)), full((1, 1)),
            full((1, D)), full((1, 1)),
        ],
        out_specs=row_spec,
        out_shape=jax.ShapeDtypeStruct((N, D), jnp.float32),
    )(x, acc[0], acc[1], deg,
      W_hp, b_hp.reshape(1, D), W_lp, b_lp.reshape(1, D),
      W_id, b_id.reshape(1, D),
      wh.reshape(1, D), bh.reshape(1, 1),
      wl.reshape(1, D), bl.reshape(1, 1),
      wi.reshape(1, D), bi.reshape(1, 1))
    return out
